# Initial kernel scaffold; baseline (speedup 1.0000x reference)
import jax
import jax.numpy as jnp
from jax.experimental import pallas as pl


def kernel(x, graph, ew, edge_index, params):
    raise NotImplementedError("write your pallas kernel here")


# speedup vs baseline: 1.0580x; 1.0580x over previous
import jax
import jax.numpy as jnp
from jax.experimental import pallas as pl


def _lin(t, p):
    return t @ p[0] + p[1]


def _layer_norm(t, g, b, eps=1e-5):
    mu = jnp.mean(t, axis=-1, keepdims=True)
    var = jnp.mean((t - mu) ** 2, axis=-1, keepdims=True)
    return (t - mu) / jnp.sqrt(var + eps) * g + b


def _final_kernel(e_in_ref, e_new_ref, g_ref, b_ref, out_ref):
    t = e_new_ref[...]
    mu = jnp.mean(t, axis=-1, keepdims=True)
    var = jnp.mean((t - mu) ** 2, axis=-1, keepdims=True)
    n = (t - mu) * jax.lax.rsqrt(var + 1e-5) * g_ref[...] + b_ref[...]
    out_ref[...] = e_in_ref[...] + jnp.maximum(n, 0.0)


def kernel(x, graph, ew, edge_index, params):
    h = _lin(_lin(x[:, None], params['node_scalar']), params['node_embed'])
    e = _lin(_lin(graph[:, None], params['edge_scalar']), params['edge_embed'])
    w = _lin(_lin(ew[:, None], params['ew_scalar']), params['ew_embed'])
    src = edge_index[0]
    dst = edge_index[1]

    lp = params['layers'][0]
    Uh = _lin(h, lp['U'])
    Uew = _lin(w, lp['U_ew'])
    Vh = _lin(h, lp['V'])[dst]
    Ah = _lin(h, lp['A'])
    Bh = _lin(h, lp['B'])
    Ce = _lin(e, lp['C'])
    Dew = _lin(w, lp['D'])
    Vew = _lin(w, lp['V_ew'])
    ew_new = Ah[dst] + Bh[src] + Dew
    e_new = ew_new + Ce
    gates = jax.nn.sigmoid(e_new)
    agg = jax.ops.segment_sum(gates * (Vh + Vew), src, num_segments=h.shape[0])
    h_new = Uh + agg
    ew_new = ew_new + Uew
    h = h + jax.nn.relu(_layer_norm(h_new, lp['norm_h'][0], lp['norm_h'][1]))
    e1 = e + jax.nn.relu(_layer_norm(e_new, lp['norm_e'][0], lp['norm_e'][1]))
    w = w + jax.nn.relu(_layer_norm(ew_new, lp['norm_ew'][0], lp['norm_ew'][1]))

    lp = params['layers'][1]
    Ah = _lin(h, lp['A'])
    Bh = _lin(h, lp['B'])
    Ce = _lin(e1, lp['C'])
    Dew = _lin(w, lp['D'])
    e_new2 = Ah[dst] + Bh[src] + Dew + Ce

    E, H = e_new2.shape
    BLK = 2000
    out = pl.pallas_call(
        _final_kernel,
        grid=(E // BLK,),
        in_specs=[
            pl.BlockSpec((BLK, H), lambda i: (i, 0)),
            pl.BlockSpec((BLK, H), lambda i: (i, 0)),
            pl.BlockSpec((H,), lambda i: (0,)),
            pl.BlockSpec((H,), lambda i: (0,)),
        ],
        out_specs=pl.BlockSpec((BLK, H), lambda i: (i, 0)),
        out_shape=jax.ShapeDtypeStruct((E, H), jnp.float32),
    )(e1, e_new2, lp['norm_e'][0], lp['norm_e'][1])
    return out


# speedup vs baseline: 3.3541x; 3.1702x over previous
import functools

import jax
import jax.numpy as jnp
from jax import lax
from jax.experimental import pallas as pl
from jax.experimental.pallas import tpu as pltpu
from jax.experimental.pallas import tpu_sc as plsc

N_NODES = 10000
N_EDGES = 160000
H = 128
NC, NS = 2, 16
NW = NC * NS
EPW = N_EDGES // NW
CHUNK = 128
NFULL = EPW // CHUNK
TAIL = EPW - NFULL * CHUNK
DROWS = 640
DSTEP = 632


@functools.lru_cache(maxsize=1)
def _sc_kernels():
    mesh = plsc.VectorSubcoreMesh(core_axis_name="c", subcore_axis_name="s",
                                  num_cores=NC, num_subcores=NS)
    cparams = pltpu.CompilerParams(needs_layout_passes=False)

    @functools.partial(
        pl.kernel,
        out_type=(jax.ShapeDtypeStruct((N_EDGES,), jnp.float32),
                  jax.ShapeDtypeStruct((N_EDGES,), jnp.float32)),
        mesh=mesh,
        scratch_types=[
            pltpu.VMEM((N_NODES,), jnp.float32),
            pltpu.VMEM((EPW,), jnp.int32),
            pltpu.VMEM((EPW,), jnp.int32),
            pltpu.VMEM((EPW,), jnp.float32),
            pltpu.VMEM((EPW,), jnp.float32),
        ],
        compiler_params=cparams,
    )
    def sc_gather_scalars(x_hbm, src_hbm, dst_hbm, sxs_hbm, sxd_hbm,
                          x_v, src_v, dst_v, os_v, od_v):
        wid = lax.axis_index("s") * NC + lax.axis_index("c")
        base = wid * EPW
        pltpu.sync_copy(x_hbm, x_v)
        pltpu.sync_copy(src_hbm.at[pl.ds(base, EPW)], src_v)
        pltpu.sync_copy(dst_hbm.at[pl.ds(base, EPW)], dst_v)

        def body(i, carry):
            st = jnp.minimum(i * 16, EPW - 16)
            os_v[pl.ds(st, 16)] = plsc.load_gather(x_v, [src_v[pl.ds(st, 16)]])
            od_v[pl.ds(st, 16)] = plsc.load_gather(x_v, [dst_v[pl.ds(st, 16)]])
            return carry

        lax.fori_loop(0, (EPW + 15) // 16, body, 0)
        pltpu.sync_copy(os_v, sxs_hbm.at[pl.ds(base, EPW)])
        pltpu.sync_copy(od_v, sxd_hbm.at[pl.ds(base, EPW)])

    @functools.partial(
        pl.kernel,
        out_type=jax.ShapeDtypeStruct((NC * N_NODES, H), jnp.float32),
        mesh=mesh,
        scratch_types=[
            pltpu.VMEM((CHUNK,), jnp.int32),
            pltpu.VMEM((TAIL,), jnp.int32),
            pltpu.VMEM((CHUNK, H), jnp.float32),
            pltpu.VMEM((TAIL, H), jnp.float32),
            pltpu.VMEM_SHARED((N_NODES, H), jnp.float32),
        ],
        compiler_params=cparams,
    )
    def sc_segment_sum(msg_hbm, src_hbm, zeros_hbm, out_hbm,
                       idx_v, idxt_v, msg_v, msgt_v, acc_sh):
        cid = lax.axis_index("c")
        sid = lax.axis_index("s")
        wid = sid * NC + cid
        base = wid * EPW
        doff = jnp.minimum(sid * DSTEP, N_NODES - DROWS)

        pltpu.sync_copy(zeros_hbm, acc_sh.at[pl.ds(doff, DROWS)])
        plsc.subcore_barrier()

        def sbody(j, carry):
            cb = base + j * CHUNK
            pltpu.sync_copy(src_hbm.at[pl.ds(cb, CHUNK)], idx_v)
            pltpu.sync_copy(msg_hbm.at[pl.ds(cb, CHUNK)], msg_v)
            pltpu.sync_copy(msg_v, acc_sh.at[idx_v], add=True)
            return carry

        lax.fori_loop(0, NFULL, sbody, 0)
        tb = base + NFULL * CHUNK
        pltpu.sync_copy(src_hbm.at[pl.ds(tb, TAIL)], idxt_v)
        pltpu.sync_copy(msg_hbm.at[pl.ds(tb, TAIL)], msgt_v)
        pltpu.sync_copy(msgt_v, acc_sh.at[idxt_v], add=True)

        plsc.subcore_barrier()
        pltpu.sync_copy(acc_sh.at[pl.ds(doff, DROWS)],
                        out_hbm.at[pl.ds(cid * N_NODES + doff, DROWS)])

    @functools.partial(
        pl.kernel,
        out_type=(jax.ShapeDtypeStruct((N_EDGES, H), jnp.float32),
                  jax.ShapeDtypeStruct((N_EDGES, H), jnp.float32)),
        mesh=mesh,
        scratch_types=[
            pltpu.VMEM((EPW,), jnp.int32),
            pltpu.VMEM((EPW,), jnp.int32),
            pltpu.VMEM((CHUNK, H), jnp.float32),
            pltpu.VMEM((CHUNK, H), jnp.float32),
            pltpu.SemaphoreType.DMA,
            pltpu.SemaphoreType.DMA,
        ],
        compiler_params=cparams,
    )
    def sc_gather_rows(ta_hbm, tb_hbm, dst_hbm, src_hbm, ga_hbm, gb_hbm,
                       idxd_v, idxs_v, bufa_v, bufb_v, sema, semb):
        cid = lax.axis_index("c")
        sid = lax.axis_index("s")
        wid = sid * NC + cid
        base = wid * EPW
        pltpu.sync_copy(dst_hbm.at[pl.ds(base, EPW)], idxd_v)
        pltpu.sync_copy(src_hbm.at[pl.ds(base, EPW)], idxs_v)

        def body(j, carry):
            off = j * CHUNK
            cpa = pltpu.async_copy(ta_hbm.at[idxd_v.at[pl.ds(off, CHUNK)]],
                                   bufa_v, sema)
            cpb = pltpu.async_copy(tb_hbm.at[idxs_v.at[pl.ds(off, CHUNK)]],
                                   bufb_v, semb)
            cpa.wait()
            cpb.wait()
            pltpu.sync_copy(bufa_v, ga_hbm.at[pl.ds(base + off, CHUNK)])
            pltpu.sync_copy(bufb_v, gb_hbm.at[pl.ds(base + off, CHUNK)])
            return carry

        lax.fori_loop(0, NFULL, body, 0)
        toff = NFULL * CHUNK
        bufa_t = bufa_v.at[pl.ds(0, TAIL)]
        bufb_t = bufb_v.at[pl.ds(0, TAIL)]
        cpa = pltpu.async_copy(ta_hbm.at[idxd_v.at[pl.ds(toff, TAIL)]],
                               bufa_t, sema)
        cpb = pltpu.async_copy(tb_hbm.at[idxs_v.at[pl.ds(toff, TAIL)]],
                               bufb_t, semb)
        cpa.wait()
        cpb.wait()
        pltpu.sync_copy(bufa_t, ga_hbm.at[pl.ds(base + toff, TAIL)])
        pltpu.sync_copy(bufb_t, gb_hbm.at[pl.ds(base + toff, TAIL)])

    return sc_gather_scalars, sc_segment_sum, sc_gather_rows


def _ln(t, g, b):
    mu = jnp.mean(t, axis=-1, keepdims=True)
    var = jnp.mean((t - mu) ** 2, axis=-1, keepdims=True)
    return (t - mu) * lax.rsqrt(var + 1e-5) * g + b


def _t1_body(sd_ref, ss_ref, gr_ref, ew_ref, c_ref, out_ref):
    sd = sd_ref[...]
    ss = ss_ref[...]
    gr = gr_ref[...]
    ew = ew_ref[...]
    c = c_ref[...]
    arg = sd * c[0:1] + ss * c[1:2] + ew * c[2:3] + gr * c[3:4] + c[4:5]
    out_ref[...] = jax.nn.sigmoid(arg) * (sd * c[5:6] + ew * c[6:7] + c[7:8])


def _t2_body(x_ref, a0_ref, a1_ref, c_ref, wa_ref, wb_ref, outa_ref, outb_ref):
    x = x_ref[...]
    c = c_ref[...]
    hn = x * c[0:1] + c[1:2] + a0_ref[...] + a1_ref[...]
    h1 = x * c[2:3] + c[3:4] + jnp.maximum(_ln(hn, c[4:5], c[5:6]), 0.0)
    outa_ref[...] = jnp.dot(h1, wa_ref[...],
                            preferred_element_type=jnp.float32) + c[6:7]
    outb_ref[...] = jnp.dot(h1, wb_ref[...],
                            preferred_element_type=jnp.float32) + c[7:8]


def _t3_body(gr_ref, ew_ref, ss_ref, sd_ref, ga_ref, gb_ref, c_ref, wc_ref,
             wd_ref, kb_ref, out_ref):
    gr = gr_ref[...]
    ew = ew_ref[...]
    ss = ss_ref[...]
    sd = sd_ref[...]
    c = c_ref[...]
    ewn1 = sd * c[0:1] + ss * c[1:2] + ew * c[2:3] + c[3:4]
    en1 = ewn1 + gr * c[4:5] + c[5:6]
    e1 = gr * c[8:9] + c[9:10] + jnp.maximum(_ln(en1, c[12:13], c[13:14]), 0.0)
    ewn1f = ewn1 + ew * c[6:7] + c[7:8]
    ew1 = ew * c[10:11] + c[11:12] + jnp.maximum(
        _ln(ewn1f, c[14:15], c[15:16]), 0.0)
    t = (ga_ref[...] + gb_ref[...] + kb_ref[...]
         + jnp.dot(e1, wc_ref[...], preferred_element_type=jnp.float32)
         + jnp.dot(ew1, wd_ref[...], preferred_element_type=jnp.float32))
    out_ref[...] = e1 + jnp.maximum(_ln(t, c[16:17], c[17:18]), 0.0)


def kernel(x, graph, ew, edge_index, params):
    p = params
    src = edge_index[0]
    dst = edge_index[1]
    l1 = p['layers'][0]
    l2 = p['layers'][1]
    sc_gather_scalars, sc_segment_sum, sc_gather_rows = _sc_kernels()

    vh = p['node_scalar'][0][0] @ p['node_embed'][0]
    chv = p['node_scalar'][1] @ p['node_embed'][0] + p['node_embed'][1]
    vev = p['edge_scalar'][0][0] @ p['edge_embed'][0]
    cev = p['edge_scalar'][1] @ p['edge_embed'][0] + p['edge_embed'][1]
    vwv = p['ew_scalar'][0][0] @ p['ew_embed'][0]
    cwv = p['ew_scalar'][1] @ p['ew_embed'][0] + p['ew_embed'][1]

    def fold(v, c, name):
        w, b = l1[name]
        return v @ w, c @ w + b

    aV, aC = fold(vh, chv, 'A')
    bV, bC = fold(vh, chv, 'B')
    dV, dC = fold(vwv, cwv, 'D')
    cV, cC = fold(vev, cev, 'C')
    vV, vC = fold(vh, chv, 'V')
    wV, wC = fold(vwv, cwv, 'V_ew')
    uV, uC = fold(vh, chv, 'U')
    uwV, uwC = fold(vwv, cwv, 'U_ew')

    sxs, sxd = sc_gather_scalars(x, src, dst)

    gr2 = graph.reshape(N_EDGES, 1)
    ew2 = ew.reshape(N_EDGES, 1)
    ss2 = sxs.reshape(N_EDGES, 1)
    sd2 = sxd.reshape(N_EDGES, 1)

    EBLK = 2000
    egrid = N_EDGES // EBLK
    c1 = jnp.stack([aV, bV, dV, cV, aC + bC + dC + cC, vV, wV, vC + wC])
    col = lambda i: (i, 0)
    full = lambda i: (0, 0)
    msg = pl.pallas_call(
        _t1_body,
        grid=(egrid,),
        in_specs=[
            pl.BlockSpec((EBLK, 1), col),
            pl.BlockSpec((EBLK, 1), col),
            pl.BlockSpec((EBLK, 1), col),
            pl.BlockSpec((EBLK, 1), col),
            pl.BlockSpec((8, H), full),
        ],
        out_specs=pl.BlockSpec((EBLK, H), col),
        out_shape=jax.ShapeDtypeStruct((N_EDGES, H), jnp.float32),
    )(sd2, ss2, gr2, ew2, c1)

    zeros = jnp.zeros((DROWS, H), jnp.float32)
    agg2 = sc_segment_sum(msg, src, zeros)

    NBLK = 1000
    ngrid = N_NODES // NBLK
    c2 = jnp.stack([uV, uC, vh, chv, l1['norm_h'][0], l1['norm_h'][1],
                    l2['A'][1], l2['B'][1]])
    x2 = x.reshape(N_NODES, 1)
    ta, tb = pl.pallas_call(
        _t2_body,
        grid=(ngrid,),
        in_specs=[
            pl.BlockSpec((NBLK, 1), col),
            pl.BlockSpec((NBLK, H), col),
            pl.BlockSpec((NBLK, H), lambda i: (i + ngrid, 0)),
            pl.BlockSpec((8, H), full),
            pl.BlockSpec((H, H), full),
            pl.BlockSpec((H, H), full),
        ],
        out_specs=[pl.BlockSpec((NBLK, H), col),
                   pl.BlockSpec((NBLK, H), col)],
        out_shape=[jax.ShapeDtypeStruct((N_NODES, H), jnp.float32),
                   jax.ShapeDtypeStruct((N_NODES, H), jnp.float32)],
    )(x2, agg2, agg2, c2, l2['A'][0], l2['B'][0])

    ga, gb = sc_gather_rows(ta, tb, dst, src)

    c3 = jnp.stack([aV, bV, dV, aC + bC + dC, cV, cC, uwV, uwC,
                    vev, cev, vwv, cwv,
                    l1['norm_e'][0], l1['norm_e'][1],
                    l1['norm_ew'][0], l1['norm_ew'][1],
                    l2['norm_e'][0], l2['norm_e'][1]])
    kb = (l2['C'][1] + l2['D'][1]).reshape(1, H)
    out = pl.pallas_call(
        _t3_body,
        grid=(egrid,),
        in_specs=[
            pl.BlockSpec((EBLK, 1), col),
            pl.BlockSpec((EBLK, 1), col),
            pl.BlockSpec((EBLK, 1), col),
            pl.BlockSpec((EBLK, 1), col),
            pl.BlockSpec((EBLK, H), col),
            pl.BlockSpec((EBLK, H), col),
            pl.BlockSpec((18, H), full),
            pl.BlockSpec((H, H), full),
            pl.BlockSpec((H, H), full),
            pl.BlockSpec((1, H), full),
        ],
        out_specs=pl.BlockSpec((EBLK, H), col),
        out_shape=jax.ShapeDtypeStruct((N_EDGES, H), jnp.float32),
    )(gr2, ew2, ss2, sd2, ga, gb, c3, l2['C'][0], l2['D'][0], kb)
    return out


# speedup vs baseline: 3.7386x; 1.1146x over previous
import functools

import jax
import jax.numpy as jnp
from jax import lax
from jax.experimental import pallas as pl
from jax.experimental.pallas import tpu as pltpu
from jax.experimental.pallas import tpu_sc as plsc

N_NODES = 10000
N_EDGES = 160000
H = 128
NC, NS = 2, 16
NW = NC * NS
EPW = N_EDGES // NW
CHUNK = 128
NFULL = EPW // CHUNK
TAIL = EPW - NFULL * CHUNK
DROWS = 640
DSTEP = 632


@functools.lru_cache(maxsize=1)
def _sc_kernels():
    mesh = plsc.VectorSubcoreMesh(core_axis_name="c", subcore_axis_name="s",
                                  num_cores=NC, num_subcores=NS)
    cparams = pltpu.CompilerParams(needs_layout_passes=False)

    @functools.partial(
        pl.kernel,
        out_type=(jax.ShapeDtypeStruct((N_EDGES,), jnp.float32),
                  jax.ShapeDtypeStruct((N_EDGES,), jnp.float32)),
        mesh=mesh,
        scratch_types=[
            pltpu.VMEM((N_NODES,), jnp.float32),
            pltpu.VMEM((EPW,), jnp.int32),
            pltpu.VMEM((EPW,), jnp.int32),
            pltpu.VMEM((EPW,), jnp.float32),
            pltpu.VMEM((EPW,), jnp.float32),
        ],
        compiler_params=cparams,
    )
    def sc_gather_scalars(x_hbm, src_hbm, dst_hbm, sxs_hbm, sxd_hbm,
                          x_v, src_v, dst_v, os_v, od_v):
        wid = lax.axis_index("s") * NC + lax.axis_index("c")
        base = wid * EPW
        pltpu.sync_copy(x_hbm, x_v)
        pltpu.sync_copy(src_hbm.at[pl.ds(base, EPW)], src_v)
        pltpu.sync_copy(dst_hbm.at[pl.ds(base, EPW)], dst_v)

        def body(i, carry):
            st = jnp.minimum(i * 16, EPW - 16)
            os_v[pl.ds(st, 16)] = plsc.load_gather(x_v, [src_v[pl.ds(st, 16)]])
            od_v[pl.ds(st, 16)] = plsc.load_gather(x_v, [dst_v[pl.ds(st, 16)]])
            return carry

        lax.fori_loop(0, (EPW + 15) // 16, body, 0)
        pltpu.sync_copy(os_v, sxs_hbm.at[pl.ds(base, EPW)])
        pltpu.sync_copy(od_v, sxd_hbm.at[pl.ds(base, EPW)])

    @functools.partial(
        pl.kernel,
        out_type=jax.ShapeDtypeStruct((NC * N_NODES, H), jnp.float32),
        mesh=mesh,
        scratch_types=[
            pltpu.VMEM((CHUNK,), jnp.int32),
            pltpu.VMEM((CHUNK,), jnp.int32),
            pltpu.VMEM((CHUNK, H), jnp.float32),
            pltpu.VMEM((CHUNK, H), jnp.float32),
            pltpu.VMEM_SHARED((N_NODES, H), jnp.float32),
            pltpu.SemaphoreType.DMA,
            pltpu.SemaphoreType.DMA,
            pltpu.SemaphoreType.DMA,
            pltpu.SemaphoreType.DMA,
            pltpu.SemaphoreType.DMA,
            pltpu.SemaphoreType.DMA,
        ],
        compiler_params=cparams,
    )
    def sc_segment_sum(msg_hbm, src_hbm, zeros_hbm, out_hbm,
                       idx0_v, idx1_v, msg0_v, msg1_v, acc_sh,
                       si0, si1, sm0, sm1, sw0, sw1):
        cid = lax.axis_index("c")
        sid = lax.axis_index("s")
        wid = sid * NC + cid
        base = wid * EPW
        doff = jnp.minimum(sid * DSTEP, N_NODES - DROWS)

        pltpu.sync_copy(zeros_hbm, acc_sh.at[pl.ds(doff, DROWS)])
        plsc.subcore_barrier()

        NCH = NFULL + 1
        idx_set = (idx0_v, idx1_v)
        msg_set = (msg0_v, msg1_v)
        si_set = (si0, si1)
        sm_set = (sm0, sm1)
        sw_set = (sw0, sw1)
        desc_g = {}
        desc_w = {}

        def issue(j):
            p = j % 2
            sz = CHUNK if j < NFULL else TAIL
            cb = base + j * CHUNK
            iv = idx_set[p] if sz == CHUNK else idx_set[p].at[pl.ds(0, TAIL)]
            mv = msg_set[p] if sz == CHUNK else msg_set[p].at[pl.ds(0, TAIL)]
            desc_g[j] = (
                pltpu.async_copy(src_hbm.at[pl.ds(cb, sz)], iv, si_set[p]),
                pltpu.async_copy(msg_hbm.at[pl.ds(cb, sz)], mv, sm_set[p]),
            )
            return iv, mv

        views = {0: issue(0)}
        for j in range(NCH):
            p = j % 2
            if j + 1 < NCH:
                if j - 1 >= 0:
                    desc_w[j - 1].wait()
                views[j + 1] = issue(j + 1)
            for d in desc_g[j]:
                d.wait()
            iv, mv = views[j]
            desc_w[j] = pltpu.async_copy(mv, acc_sh.at[iv], sw_set[p],
                                         add=True)
        desc_w[NCH - 2].wait()
        desc_w[NCH - 1].wait()

        plsc.subcore_barrier()
        pltpu.sync_copy(acc_sh.at[pl.ds(doff, DROWS)],
                        out_hbm.at[pl.ds(cid * N_NODES + doff, DROWS)])

    @functools.partial(
        pl.kernel,
        out_type=jax.ShapeDtypeStruct((N_EDGES, H), jnp.float32),
        mesh=mesh,
        scratch_types=[
            pltpu.VMEM((EPW,), jnp.int32),
            pltpu.VMEM((EPW,), jnp.int32),
            pltpu.VMEM((CHUNK, H), jnp.float32),
            pltpu.VMEM((CHUNK, H), jnp.float32),
            pltpu.VMEM((CHUNK, H), jnp.float32),
            pltpu.VMEM((CHUNK, H), jnp.float32),
            pltpu.SemaphoreType.DMA,
            pltpu.SemaphoreType.DMA,
            pltpu.SemaphoreType.DMA,
            pltpu.SemaphoreType.DMA,
            pltpu.SemaphoreType.DMA,
            pltpu.SemaphoreType.DMA,
        ],
        compiler_params=cparams,
    )
    def sc_gather_rows(ta_hbm, tb_hbm, dst_hbm, src_hbm, g_hbm,
                       idxd_v, idxs_v, bufa0_v, bufb0_v, bufa1_v, bufb1_v,
                       sa0, sb0, sa1, sb1, sw0, sw1):
        cid = lax.axis_index("c")
        sid = lax.axis_index("s")
        wid = sid * NC + cid
        base = wid * EPW
        pltpu.sync_copy(dst_hbm.at[pl.ds(base, EPW)], idxd_v)
        pltpu.sync_copy(src_hbm.at[pl.ds(base, EPW)], idxs_v)

        NCH = NFULL + 1
        ba_set = (bufa0_v, bufa1_v)
        bb_set = (bufb0_v, bufb1_v)
        sa_set = (sa0, sa1)
        sb_set = (sb0, sb1)
        sw_set = (sw0, sw1)
        desc_g = {}
        desc_w = {}

        def issue(j):
            p = j % 2
            sz = CHUNK if j < NFULL else TAIL
            off = j * CHUNK
            ba = ba_set[p] if sz == CHUNK else ba_set[p].at[pl.ds(0, TAIL)]
            bb = bb_set[p] if sz == CHUNK else bb_set[p].at[pl.ds(0, TAIL)]
            desc_g[j] = (
                pltpu.async_copy(ta_hbm.at[idxd_v.at[pl.ds(off, sz)]],
                                 ba, sa_set[p]),
                pltpu.async_copy(tb_hbm.at[idxs_v.at[pl.ds(off, sz)]],
                                 bb, sb_set[p]),
            )
            return ba, bb, sz, off

        views = {0: issue(0)}
        for j in range(NCH):
            p = j % 2
            if j + 1 < NCH:
                if j - 1 >= 0:
                    desc_w[j - 1].wait()
                views[j + 1] = issue(j + 1)
            for d in desc_g[j]:
                d.wait()
            ba, bb, sz, off = views[j]

            def abody(r, c2, _ba=ba, _bb=bb):
                for cc in range(H // 16):
                    sl = pl.ds(cc * 16, 16)
                    _ba[r, sl] = _ba[r, sl] + _bb[r, sl]
                return c2

            lax.fori_loop(0, sz, abody, 0)
            desc_w[j] = pltpu.async_copy(
                ba, g_hbm.at[pl.ds(base + off, sz)], sw_set[p])
        desc_w[NCH - 2].wait()
        desc_w[NCH - 1].wait()

    return sc_gather_scalars, sc_segment_sum, sc_gather_rows


def _ln(t, g, b):
    mu = jnp.mean(t, axis=-1, keepdims=True)
    var = jnp.mean((t - mu) ** 2, axis=-1, keepdims=True)
    return (t - mu) * lax.rsqrt(var + 1e-5) * g + b


def _t1_body(sd_ref, ss_ref, gr_ref, ew_ref, c_ref, out_ref):
    sd = sd_ref[...]
    ss = ss_ref[...]
    gr = gr_ref[...]
    ew = ew_ref[...]
    c = c_ref[...]
    arg = sd * c[0:1] + ss * c[1:2] + ew * c[2:3] + gr * c[3:4] + c[4:5]
    out_ref[...] = jax.nn.sigmoid(arg) * (sd * c[5:6] + ew * c[6:7] + c[7:8])


def _t2_body(x_ref, a0_ref, a1_ref, c_ref, wa_ref, wb_ref, outa_ref, outb_ref):
    x = x_ref[...]
    c = c_ref[...]
    hn = x * c[0:1] + c[1:2] + a0_ref[...] + a1_ref[...]
    h1 = x * c[2:3] + c[3:4] + jnp.maximum(_ln(hn, c[4:5], c[5:6]), 0.0)
    outa_ref[...] = jnp.dot(h1, wa_ref[...],
                            preferred_element_type=jnp.float32) + c[6:7]
    outb_ref[...] = jnp.dot(h1, wb_ref[...],
                            preferred_element_type=jnp.float32) + c[7:8]


def _t3_body(gr_ref, ew_ref, ss_ref, sd_ref, g_ref, c_ref, wc_ref,
             wd_ref, kb_ref, out_ref):
    gr = gr_ref[...]
    ew = ew_ref[...]
    ss = ss_ref[...]
    sd = sd_ref[...]
    c = c_ref[...]
    ewn1 = sd * c[0:1] + ss * c[1:2] + ew * c[2:3] + c[3:4]
    en1 = ewn1 + gr * c[4:5] + c[5:6]
    e1 = gr * c[8:9] + c[9:10] + jnp.maximum(_ln(en1, c[12:13], c[13:14]), 0.0)
    ewn1f = ewn1 + ew * c[6:7] + c[7:8]
    ew1 = ew * c[10:11] + c[11:12] + jnp.maximum(
        _ln(ewn1f, c[14:15], c[15:16]), 0.0)
    t = (g_ref[...] + kb_ref[...]
         + jnp.dot(e1, wc_ref[...], preferred_element_type=jnp.float32)
         + jnp.dot(ew1, wd_ref[...], preferred_element_type=jnp.float32))
    out_ref[...] = e1 + jnp.maximum(_ln(t, c[16:17], c[17:18]), 0.0)


def kernel(x, graph, ew, edge_index, params):
    p = params
    src = edge_index[0]
    dst = edge_index[1]
    l1 = p['layers'][0]
    l2 = p['layers'][1]
    sc_gather_scalars, sc_segment_sum, sc_gather_rows = _sc_kernels()

    vh = p['node_scalar'][0][0] @ p['node_embed'][0]
    chv = p['node_scalar'][1] @ p['node_embed'][0] + p['node_embed'][1]
    vev = p['edge_scalar'][0][0] @ p['edge_embed'][0]
    cev = p['edge_scalar'][1] @ p['edge_embed'][0] + p['edge_embed'][1]
    vwv = p['ew_scalar'][0][0] @ p['ew_embed'][0]
    cwv = p['ew_scalar'][1] @ p['ew_embed'][0] + p['ew_embed'][1]

    def fold(v, c, name):
        w, b = l1[name]
        return v @ w, c @ w + b

    aV, aC = fold(vh, chv, 'A')
    bV, bC = fold(vh, chv, 'B')
    dV, dC = fold(vwv, cwv, 'D')
    cV, cC = fold(vev, cev, 'C')
    vV, vC = fold(vh, chv, 'V')
    wV, wC = fold(vwv, cwv, 'V_ew')
    uV, uC = fold(vh, chv, 'U')
    uwV, uwC = fold(vwv, cwv, 'U_ew')

    sxs, sxd = sc_gather_scalars(x, src, dst)

    gr2 = graph.reshape(N_EDGES, 1)
    ew2 = ew.reshape(N_EDGES, 1)
    ss2 = sxs.reshape(N_EDGES, 1)
    sd2 = sxd.reshape(N_EDGES, 1)

    EBLK = 2000
    egrid = N_EDGES // EBLK
    c1 = jnp.stack([aV, bV, dV, cV, aC + bC + dC + cC, vV, wV, vC + wC])
    col = lambda i: (i, 0)
    full = lambda i: (0, 0)
    msg = pl.pallas_call(
        _t1_body,
        grid=(egrid,),
        in_specs=[
            pl.BlockSpec((EBLK, 1), col),
            pl.BlockSpec((EBLK, 1), col),
            pl.BlockSpec((EBLK, 1), col),
            pl.BlockSpec((EBLK, 1), col),
            pl.BlockSpec((8, H), full),
        ],
        out_specs=pl.BlockSpec((EBLK, H), col),
        out_shape=jax.ShapeDtypeStruct((N_EDGES, H), jnp.float32),
    )(sd2, ss2, gr2, ew2, c1)

    zeros = jnp.zeros((DROWS, H), jnp.float32)
    agg2 = sc_segment_sum(msg, src, zeros)

    NBLK = 1000
    ngrid = N_NODES // NBLK
    c2 = jnp.stack([uV, uC, vh, chv, l1['norm_h'][0], l1['norm_h'][1],
                    l2['A'][1], l2['B'][1]])
    x2 = x.reshape(N_NODES, 1)
    ta, tb = pl.pallas_call(
        _t2_body,
        grid=(ngrid,),
        in_specs=[
            pl.BlockSpec((NBLK, 1), col),
            pl.BlockSpec((NBLK, H), col),
            pl.BlockSpec((NBLK, H), lambda i: (i + ngrid, 0)),
            pl.BlockSpec((8, H), full),
            pl.BlockSpec((H, H), full),
            pl.BlockSpec((H, H), full),
        ],
        out_specs=[pl.BlockSpec((NBLK, H), col),
                   pl.BlockSpec((NBLK, H), col)],
        out_shape=[jax.ShapeDtypeStruct((N_NODES, H), jnp.float32),
                   jax.ShapeDtypeStruct((N_NODES, H), jnp.float32)],
    )(x2, agg2, agg2, c2, l2['A'][0], l2['B'][0])

    g = sc_gather_rows(ta, tb, dst, src)

    c3 = jnp.stack([aV, bV, dV, aC + bC + dC, cV, cC, uwV, uwC,
                    vev, cev, vwv, cwv,
                    l1['norm_e'][0], l1['norm_e'][1],
                    l1['norm_ew'][0], l1['norm_ew'][1],
                    l2['norm_e'][0], l2['norm_e'][1]])
    kb = (l2['C'][1] + l2['D'][1]).reshape(1, H)
    out = pl.pallas_call(
        _t3_body,
        grid=(egrid,),
        in_specs=[
            pl.BlockSpec((EBLK, 1), col),
            pl.BlockSpec((EBLK, 1), col),
            pl.BlockSpec((EBLK, 1), col),
            pl.BlockSpec((EBLK, 1), col),
            pl.BlockSpec((EBLK, H), col),
            pl.BlockSpec((18, H), full),
            pl.BlockSpec((H, H), full),
            pl.BlockSpec((H, H), full),
            pl.BlockSpec((1, H), full),
        ],
        out_specs=pl.BlockSpec((EBLK, H), col),
        out_shape=jax.ShapeDtypeStruct((N_EDGES, H), jnp.float32),
    )(gr2, ew2, ss2, sd2, g, c3, l2['C'][0], l2['D'][0], kb)
    return out


# speedup vs baseline: 3.7399x; 1.0003x over previous
import functools

import jax
import jax.numpy as jnp
from jax import lax
from jax.experimental import pallas as pl
from jax.experimental.pallas import tpu as pltpu
from jax.experimental.pallas import tpu_sc as plsc

N_NODES = 10000
N_EDGES = 160000
H = 128
NC, NS = 2, 16
NW = NC * NS
EPW = N_EDGES // NW
CHUNK = 128
NFULL = EPW // CHUNK
TAIL = EPW - NFULL * CHUNK
DROWS = 640
DSTEP = 632


@functools.lru_cache(maxsize=1)
def _sc_kernels():
    mesh = plsc.VectorSubcoreMesh(core_axis_name="c", subcore_axis_name="s",
                                  num_cores=NC, num_subcores=NS)
    cparams = pltpu.CompilerParams(needs_layout_passes=False,
                                   skip_device_barrier=True)

    @functools.partial(
        pl.kernel,
        out_type=(jax.ShapeDtypeStruct((N_EDGES,), jnp.float32),
                  jax.ShapeDtypeStruct((N_EDGES,), jnp.float32)),
        mesh=mesh,
        scratch_types=[
            pltpu.VMEM((N_NODES,), jnp.float32),
            pltpu.VMEM((EPW,), jnp.int32),
            pltpu.VMEM((EPW,), jnp.int32),
            pltpu.VMEM((EPW,), jnp.float32),
            pltpu.VMEM((EPW,), jnp.float32),
        ],
        compiler_params=cparams,
    )
    def sc_gather_scalars(x_hbm, src_hbm, dst_hbm, sxs_hbm, sxd_hbm,
                          x_v, src_v, dst_v, os_v, od_v):
        wid = lax.axis_index("s") * NC + lax.axis_index("c")
        base = wid * EPW
        pltpu.sync_copy(x_hbm, x_v)
        pltpu.sync_copy(src_hbm.at[pl.ds(base, EPW)], src_v)
        pltpu.sync_copy(dst_hbm.at[pl.ds(base, EPW)], dst_v)

        def body(i, carry):
            st = jnp.minimum(i * 16, EPW - 16)
            os_v[pl.ds(st, 16)] = plsc.load_gather(x_v, [src_v[pl.ds(st, 16)]])
            od_v[pl.ds(st, 16)] = plsc.load_gather(x_v, [dst_v[pl.ds(st, 16)]])
            return carry

        lax.fori_loop(0, (EPW + 15) // 16, body, 0)
        pltpu.sync_copy(os_v, sxs_hbm.at[pl.ds(base, EPW)])
        pltpu.sync_copy(od_v, sxd_hbm.at[pl.ds(base, EPW)])

    @functools.partial(
        pl.kernel,
        out_type=jax.ShapeDtypeStruct((NC * N_NODES, H), jnp.float32),
        mesh=mesh,
        scratch_types=[
            pltpu.VMEM((CHUNK,), jnp.int32),
            pltpu.VMEM((CHUNK,), jnp.int32),
            pltpu.VMEM((CHUNK, H), jnp.float32),
            pltpu.VMEM((CHUNK, H), jnp.float32),
            pltpu.VMEM_SHARED((N_NODES, H), jnp.float32),
            pltpu.SemaphoreType.DMA,
            pltpu.SemaphoreType.DMA,
            pltpu.SemaphoreType.DMA,
            pltpu.SemaphoreType.DMA,
            pltpu.SemaphoreType.DMA,
            pltpu.SemaphoreType.DMA,
        ],
        compiler_params=cparams,
    )
    def sc_segment_sum(msg_hbm, src_hbm, zeros_hbm, out_hbm,
                       idx0_v, idx1_v, msg0_v, msg1_v, acc_sh,
                       si0, si1, sm0, sm1, sw0, sw1):
        cid = lax.axis_index("c")
        sid = lax.axis_index("s")
        wid = sid * NC + cid
        base = wid * EPW
        doff = jnp.minimum(sid * DSTEP, N_NODES - DROWS)

        pltpu.sync_copy(zeros_hbm, acc_sh.at[pl.ds(doff, DROWS)])
        plsc.subcore_barrier()

        NCH = NFULL + 1
        idx_set = (idx0_v, idx1_v)
        msg_set = (msg0_v, msg1_v)
        si_set = (si0, si1)
        sm_set = (sm0, sm1)
        sw_set = (sw0, sw1)
        desc_g = {}
        desc_w = {}

        def issue(j):
            p = j % 2
            sz = CHUNK if j < NFULL else TAIL
            cb = base + j * CHUNK
            iv = idx_set[p] if sz == CHUNK else idx_set[p].at[pl.ds(0, TAIL)]
            mv = msg_set[p] if sz == CHUNK else msg_set[p].at[pl.ds(0, TAIL)]
            desc_g[j] = (
                pltpu.async_copy(src_hbm.at[pl.ds(cb, sz)], iv, si_set[p]),
                pltpu.async_copy(msg_hbm.at[pl.ds(cb, sz)], mv, sm_set[p]),
            )
            return iv, mv

        views = {0: issue(0)}
        for j in range(NCH):
            p = j % 2
            if j + 1 < NCH:
                if j - 1 >= 0:
                    desc_w[j - 1].wait()
                views[j + 1] = issue(j + 1)
            for d in desc_g[j]:
                d.wait()
            iv, mv = views[j]
            desc_w[j] = pltpu.async_copy(mv, acc_sh.at[iv], sw_set[p],
                                         add=True)
        desc_w[NCH - 2].wait()
        desc_w[NCH - 1].wait()

        plsc.subcore_barrier()
        pltpu.sync_copy(acc_sh.at[pl.ds(doff, DROWS)],
                        out_hbm.at[pl.ds(cid * N_NODES + doff, DROWS)])

    @functools.partial(
        pl.kernel,
        out_type=jax.ShapeDtypeStruct((N_EDGES, H), jnp.float32),
        mesh=mesh,
        scratch_types=[
            pltpu.VMEM((EPW,), jnp.int32),
            pltpu.VMEM((EPW,), jnp.int32),
            pltpu.VMEM((CHUNK, H), jnp.float32),
            pltpu.VMEM((CHUNK, H), jnp.float32),
            pltpu.VMEM((CHUNK, H), jnp.float32),
            pltpu.VMEM((CHUNK, H), jnp.float32),
            pltpu.SemaphoreType.DMA,
            pltpu.SemaphoreType.DMA,
            pltpu.SemaphoreType.DMA,
            pltpu.SemaphoreType.DMA,
            pltpu.SemaphoreType.DMA,
            pltpu.SemaphoreType.DMA,
        ],
        compiler_params=cparams,
    )
    def sc_gather_rows(ta_hbm, tb_hbm, dst_hbm, src_hbm, g_hbm,
                       idxd_v, idxs_v, bufa0_v, bufb0_v, bufa1_v, bufb1_v,
                       sa0, sb0, sa1, sb1, sw0, sw1):
        cid = lax.axis_index("c")
        sid = lax.axis_index("s")
        wid = sid * NC + cid
        base = wid * EPW
        pltpu.sync_copy(dst_hbm.at[pl.ds(base, EPW)], idxd_v)
        pltpu.sync_copy(src_hbm.at[pl.ds(base, EPW)], idxs_v)

        NCH = NFULL + 1
        ba_set = (bufa0_v, bufa1_v)
        bb_set = (bufb0_v, bufb1_v)
        sa_set = (sa0, sa1)
        sb_set = (sb0, sb1)
        sw_set = (sw0, sw1)
        desc_g = {}
        desc_w = {}

        def issue(j):
            p = j % 2
            sz = CHUNK if j < NFULL else TAIL
            off = j * CHUNK
            ba = ba_set[p] if sz == CHUNK else ba_set[p].at[pl.ds(0, TAIL)]
            bb = bb_set[p] if sz == CHUNK else bb_set[p].at[pl.ds(0, TAIL)]
            desc_g[j] = (
                pltpu.async_copy(ta_hbm.at[idxd_v.at[pl.ds(off, sz)]],
                                 ba, sa_set[p]),
                pltpu.async_copy(tb_hbm.at[idxs_v.at[pl.ds(off, sz)]],
                                 bb, sb_set[p]),
            )
            return ba, bb, sz, off

        views = {0: issue(0)}
        for j in range(NCH):
            p = j % 2
            if j + 1 < NCH:
                if j - 1 >= 0:
                    desc_w[j - 1].wait()
                views[j + 1] = issue(j + 1)
            for d in desc_g[j]:
                d.wait()
            ba, bb, sz, off = views[j]

            def abody(r, c2, _ba=ba, _bb=bb):
                for cc in range(H // 16):
                    sl = pl.ds(cc * 16, 16)
                    _ba[r, sl] = _ba[r, sl] + _bb[r, sl]
                return c2

            lax.fori_loop(0, sz, abody, 0)
            desc_w[j] = pltpu.async_copy(
                ba, g_hbm.at[pl.ds(base + off, sz)], sw_set[p])
        desc_w[NCH - 2].wait()
        desc_w[NCH - 1].wait()

    return sc_gather_scalars, sc_segment_sum, sc_gather_rows


def _ln(t, g, b):
    mu = jnp.mean(t, axis=-1, keepdims=True)
    var = jnp.mean((t - mu) ** 2, axis=-1, keepdims=True)
    return (t - mu) * lax.rsqrt(var + 1e-5) * g + b


def _t1_body(sd_ref, ss_ref, gr_ref, ew_ref, c_ref, out_ref):
    sd = sd_ref[...]
    ss = ss_ref[...]
    gr = gr_ref[...]
    ew = ew_ref[...]
    c = c_ref[...]
    arg = sd * c[0:1] + ss * c[1:2] + ew * c[2:3] + gr * c[3:4] + c[4:5]
    out_ref[...] = jax.nn.sigmoid(arg) * (sd * c[5:6] + ew * c[6:7] + c[7:8])


def _t2_body(x_ref, a0_ref, a1_ref, c_ref, wa_ref, wb_ref, outa_ref, outb_ref):
    x = x_ref[...]
    c = c_ref[...]
    hn = x * c[0:1] + c[1:2] + a0_ref[...] + a1_ref[...]
    h1 = x * c[2:3] + c[3:4] + jnp.maximum(_ln(hn, c[4:5], c[5:6]), 0.0)
    outa_ref[...] = jnp.dot(h1, wa_ref[...],
                            preferred_element_type=jnp.float32) + c[6:7]
    outb_ref[...] = jnp.dot(h1, wb_ref[...],
                            preferred_element_type=jnp.float32) + c[7:8]


def _t3_body(gr_ref, ew_ref, ss_ref, sd_ref, g_ref, c_ref, wc_ref,
             wd_ref, kb_ref, out_ref):
    gr = gr_ref[...]
    ew = ew_ref[...]
    ss = ss_ref[...]
    sd = sd_ref[...]
    c = c_ref[...]
    ewn1 = sd * c[0:1] + ss * c[1:2] + ew * c[2:3] + c[3:4]
    en1 = ewn1 + gr * c[4:5] + c[5:6]
    e1 = gr * c[8:9] + c[9:10] + jnp.maximum(_ln(en1, c[12:13], c[13:14]), 0.0)
    ewn1f = ewn1 + ew * c[6:7] + c[7:8]
    ew1 = ew * c[10:11] + c[11:12] + jnp.maximum(
        _ln(ewn1f, c[14:15], c[15:16]), 0.0)
    t = (g_ref[...] + kb_ref[...]
         + jnp.dot(e1, wc_ref[...], preferred_element_type=jnp.float32)
         + jnp.dot(ew1, wd_ref[...], preferred_element_type=jnp.float32))
    out_ref[...] = e1 + jnp.maximum(_ln(t, c[16:17], c[17:18]), 0.0)


def kernel(x, graph, ew, edge_index, params):
    p = params
    src = edge_index[0]
    dst = edge_index[1]
    l1 = p['layers'][0]
    l2 = p['layers'][1]
    sc_gather_scalars, sc_segment_sum, sc_gather_rows = _sc_kernels()

    vh = p['node_scalar'][0][0] @ p['node_embed'][0]
    chv = p['node_scalar'][1] @ p['node_embed'][0] + p['node_embed'][1]
    vev = p['edge_scalar'][0][0] @ p['edge_embed'][0]
    cev = p['edge_scalar'][1] @ p['edge_embed'][0] + p['edge_embed'][1]
    vwv = p['ew_scalar'][0][0] @ p['ew_embed'][0]
    cwv = p['ew_scalar'][1] @ p['ew_embed'][0] + p['ew_embed'][1]

    def fold(v, c, name):
        w, b = l1[name]
        return v @ w, c @ w + b

    aV, aC = fold(vh, chv, 'A')
    bV, bC = fold(vh, chv, 'B')
    dV, dC = fold(vwv, cwv, 'D')
    cV, cC = fold(vev, cev, 'C')
    vV, vC = fold(vh, chv, 'V')
    wV, wC = fold(vwv, cwv, 'V_ew')
    uV, uC = fold(vh, chv, 'U')
    uwV, uwC = fold(vwv, cwv, 'U_ew')

    sxs, sxd = sc_gather_scalars(x, src, dst)

    gr2 = graph.reshape(N_EDGES, 1)
    ew2 = ew.reshape(N_EDGES, 1)
    ss2 = sxs.reshape(N_EDGES, 1)
    sd2 = sxd.reshape(N_EDGES, 1)

    EBLK = 2000
    egrid = N_EDGES // EBLK
    c1 = jnp.stack([aV, bV, dV, cV, aC + bC + dC + cC, vV, wV, vC + wC])
    col = lambda i: (i, 0)
    full = lambda i: (0, 0)
    msg = pl.pallas_call(
        _t1_body,
        grid=(egrid,),
        in_specs=[
            pl.BlockSpec((EBLK, 1), col),
            pl.BlockSpec((EBLK, 1), col),
            pl.BlockSpec((EBLK, 1), col),
            pl.BlockSpec((EBLK, 1), col),
            pl.BlockSpec((8, H), full),
        ],
        out_specs=pl.BlockSpec((EBLK, H), col),
        out_shape=jax.ShapeDtypeStruct((N_EDGES, H), jnp.float32),
    )(sd2, ss2, gr2, ew2, c1)

    zeros = jnp.zeros((DROWS, H), jnp.float32)
    agg2 = sc_segment_sum(msg, src, zeros)

    NBLK = 1000
    ngrid = N_NODES // NBLK
    c2 = jnp.stack([uV, uC, vh, chv, l1['norm_h'][0], l1['norm_h'][1],
                    l2['A'][1], l2['B'][1]])
    x2 = x.reshape(N_NODES, 1)
    ta, tb = pl.pallas_call(
        _t2_body,
        grid=(ngrid,),
        in_specs=[
            pl.BlockSpec((NBLK, 1), col),
            pl.BlockSpec((NBLK, H), col),
            pl.BlockSpec((NBLK, H), lambda i: (i + ngrid, 0)),
            pl.BlockSpec((8, H), full),
            pl.BlockSpec((H, H), full),
            pl.BlockSpec((H, H), full),
        ],
        out_specs=[pl.BlockSpec((NBLK, H), col),
                   pl.BlockSpec((NBLK, H), col)],
        out_shape=[jax.ShapeDtypeStruct((N_NODES, H), jnp.float32),
                   jax.ShapeDtypeStruct((N_NODES, H), jnp.float32)],
    )(x2, agg2, agg2, c2, l2['A'][0], l2['B'][0])

    g = sc_gather_rows(ta, tb, dst, src)

    c3 = jnp.stack([aV, bV, dV, aC + bC + dC, cV, cC, uwV, uwC,
                    vev, cev, vwv, cwv,
                    l1['norm_e'][0], l1['norm_e'][1],
                    l1['norm_ew'][0], l1['norm_ew'][1],
                    l2['norm_e'][0], l2['norm_e'][1]])
    kb = (l2['C'][1] + l2['D'][1]).reshape(1, H)
    out = pl.pallas_call(
        _t3_body,
        grid=(egrid,),
        in_specs=[
            pl.BlockSpec((EBLK, 1), col),
            pl.BlockSpec((EBLK, 1), col),
            pl.BlockSpec((EBLK, 1), col),
            pl.BlockSpec((EBLK, 1), col),
            pl.BlockSpec((EBLK, H), col),
            pl.BlockSpec((18, H), full),
            pl.BlockSpec((H, H), full),
            pl.BlockSpec((H, H), full),
            pl.BlockSpec((1, H), full),
        ],
        out_specs=pl.BlockSpec((EBLK, H), col),
        out_shape=jax.ShapeDtypeStruct((N_EDGES, H), jnp.float32),
    )(gr2, ew2, ss2, sd2, g, c3, l2['C'][0], l2['D'][0], kb)
    return out
